# Initial kernel scaffold; baseline (speedup 1.0000x reference)
import jax
import jax.numpy as jnp
from jax.experimental import pallas as pl


def kernel(s_feats, q_points, s_points, neighbor_indices, weights, kernel_points):
    raise NotImplementedError("write your pallas kernel here")


# speedup vs baseline: 2.4666x; 2.4666x over previous
import functools

import jax
import jax.numpy as jnp
from jax import lax
from jax.experimental import pallas as pl
from jax.experimental.pallas import tpu as pltpu
from jax.experimental.pallas import tpu_sc as plsc

K_SIZE = 15
KP = 16
IN_DIM = 128
OUT_DIM = 128
SIGMA = 2.0
DIM = 3
N_POINTS = 10000
N_NEIGHBORS = 32
AUX = 8

BQ = 200

_SC_INFO = plsc.get_sparse_core_info()
NW = _SC_INFO.num_cores * _SC_INFO.num_subcores
TOTAL_ROWS = N_POINTS * N_NEIGHBORS
PER_W = TOTAL_ROWS // NW
CH = 80
N_CH = PER_W // CH
LANES = 16


def _sc_gather_body(feats_hbm, aux_hbm, idx_hbm, f_out, a_out,
                    aux_tab, idx_v, rows_v, aux_v, sem):
    wid = lax.axis_index("s") * _SC_INFO.num_cores + lax.axis_index("c")
    base = wid * PER_W

    pltpu.sync_copy(aux_hbm, aux_tab)
    zero = jnp.zeros((LANES,), jnp.float32)
    for j in range(CH * AUX // LANES):
        aux_v[pl.ds(j * LANES, LANES)] = zero

    def chunk(i, carry):
        off = base + i * CH
        pltpu.sync_copy(idx_hbm.at[pl.ds(off, CH)], idx_v)
        cp = pltpu.async_copy(feats_hbm.at[idx_v], rows_v, sem)
        for v in range(CH // LANES):
            idx16 = idx_v[pl.ds(v * LANES, LANES)] * AUX
            dst = (lax.iota(jnp.int32, LANES) + v * LANES) * AUX
            for c in range(5):
                vals = plsc.load_gather(aux_tab, [idx16 + c])
                plsc.store_scatter(aux_v, [dst + c], vals)
        cp.wait()
        pltpu.sync_copy(rows_v, f_out.at[pl.ds(off, CH)])
        pltpu.sync_copy(aux_v, a_out.at[pl.ds(off * AUX, CH * AUX)])
        return carry

    lax.fori_loop(0, N_CH, chunk, 0)


def _sc_gather(feats, aux_flat, idx_flat):
    mesh = plsc.VectorSubcoreMesh(core_axis_name="c", subcore_axis_name="s")
    return pl.kernel(
        _sc_gather_body,
        out_type=(
            jax.ShapeDtypeStruct((TOTAL_ROWS, IN_DIM), jnp.float32),
            jax.ShapeDtypeStruct((TOTAL_ROWS * AUX,), jnp.float32),
        ),
        mesh=mesh,
        compiler_params=pltpu.CompilerParams(needs_layout_passes=False),
        scratch_types=[
            pltpu.VMEM((N_POINTS * AUX,), jnp.float32),
            pltpu.VMEM((CH,), jnp.int32),
            pltpu.VMEM((CH, IN_DIM), jnp.float32),
            pltpu.VMEM((CH * AUX,), jnp.float32),
            pltpu.SemaphoreType.DMA,
        ],
    )(feats, aux_flat, idx_flat)


def _tc_body(f_ref, aux_ref, qp_ref, kp_ref, w_ref, out_ref):
    nf = f_ref[...]
    qp = qp_ref[...]

    rel = aux_ref[...] - qp[:, None, :]
    rc = rel[:, :, :4]
    r2 = jnp.sum(rc * rc, axis=2, keepdims=True)

    cross = jax.lax.dot_general(
        rel, kp_ref[...],
        dimension_numbers=(((2,), (0,)), ((), ())),
        precision=jax.lax.Precision.HIGHEST,
        preferred_element_type=jnp.float32,
    )
    sq = jnp.maximum(r2 + cross, 0.0)
    wk = jnp.maximum(1.0 - jnp.sqrt(sq) * (1.0 / SIGMA), 0.0)

    wf = jax.lax.dot_general(
        wk, nf,
        dimension_numbers=(((1,), (1,)), ((0,), (0,))),
        preferred_element_type=jnp.float32,
    )

    ind = aux_ref[:, :, 4]
    cnt = jnp.maximum(jnp.sum(ind, axis=1, keepdims=True), 1.0)

    acc = jnp.dot(wf.reshape(BQ, KP * IN_DIM), w_ref[...],
                  preferred_element_type=jnp.float32)
    out_ref[...] = acc / cnt


@jax.jit
def _run(s_feats, aux_flat, q_points, neighbor_indices, weights_pad, kp_packed):
    f_g, a_g = _sc_gather(s_feats, aux_flat, neighbor_indices.reshape(-1))
    f_g = f_g.reshape(N_POINTS, N_NEIGHBORS, IN_DIM)
    a_g = a_g.reshape(N_POINTS, N_NEIGHBORS, AUX)

    grid = N_POINTS // BQ
    return pl.pallas_call(
        _tc_body,
        grid=(grid,),
        in_specs=[
            pl.BlockSpec((BQ, N_NEIGHBORS, IN_DIM), lambda i: (i, 0, 0)),
            pl.BlockSpec((BQ, N_NEIGHBORS, AUX), lambda i: (i, 0, 0)),
            pl.BlockSpec((BQ, 8), lambda i: (i, 0)),
            pl.BlockSpec((8, KP), lambda i: (0, 0)),
            pl.BlockSpec((KP * IN_DIM, OUT_DIM), lambda i: (0, 0)),
        ],
        out_specs=pl.BlockSpec((BQ, OUT_DIM), lambda i: (i, 0)),
        out_shape=jax.ShapeDtypeStruct((N_POINTS, OUT_DIM), jnp.float32),
    )(f_g, a_g, q_points, kp_packed, weights_pad)


def kernel(s_feats, q_points, s_points, neighbor_indices, weights, kernel_points):
    ones = jnp.ones((N_POINTS, 1), jnp.float32)
    ind = (jnp.sum(s_feats, axis=1, keepdims=True) > 0.0).astype(jnp.float32)
    zeros = jnp.zeros((N_POINTS, AUX - DIM - 2), jnp.float32)
    aux = jnp.concatenate([s_points, ones, ind, zeros], axis=1)
    qp_pad = jnp.pad(q_points, ((0, 0), (0, 5)))
    kp_packed = jnp.zeros((8, KP), jnp.float32)
    kp_packed = kp_packed.at[:3, :K_SIZE].set(-2.0 * kernel_points.T)
    kp_packed = kp_packed.at[3, :K_SIZE].set(
        jnp.sum(kernel_points ** 2, axis=1) - 1.0)
    kp_packed = kp_packed.at[3, K_SIZE].set(-1e9)
    weights_pad = jnp.concatenate(
        [weights, jnp.zeros((KP - K_SIZE, IN_DIM, OUT_DIM), jnp.float32)],
        axis=0).reshape(KP * IN_DIM, OUT_DIM)
    return _run(s_feats, aux.reshape(-1), qp_pad, neighbor_indices,
                weights_pad, kp_packed)


# speedup vs baseline: 3.1296x; 1.2688x over previous
import functools

import jax
import jax.numpy as jnp
from jax import lax
from jax.experimental import pallas as pl
from jax.experimental.pallas import tpu as pltpu
from jax.experimental.pallas import tpu_sc as plsc

K_SIZE = 15
KP = 16
IN_DIM = 128
OUT_DIM = 128
SIGMA = 2.0
DIM = 3
N_POINTS = 10000
N_NEIGHBORS = 32
AUX = 5
AUXO = 8

BQ = 200

_SC_INFO = plsc.get_sparse_core_info()
NW = _SC_INFO.num_cores * _SC_INFO.num_subcores
TOTAL_ROWS = N_POINTS * N_NEIGHBORS
PER_W = TOTAL_ROWS // NW
CH = 80
NB = 5
N_IT = PER_W // (CH * NB)
CHUNKS_W = PER_W // CH
LANES = 16


def _sc_gather_body(feats_hbm, aux_hbm, idx_hbm, f_out, a_out,
                    aux_tab, idx_all, rows_v, aux_v, g_sem, wb_sem):
    wid = lax.axis_index("s") * _SC_INFO.num_cores + lax.axis_index("c")
    base = wid * PER_W

    pltpu.sync_copy(aux_hbm, aux_tab)
    pltpu.sync_copy(idx_hbm.at[pl.ds(base, PER_W)], idx_all)
    zero = jnp.zeros((LANES,), jnp.float32)
    for z in range(CH * NB * AUXO // LANES):
        aux_v[pl.ds(z * LANES, LANES)] = zero

    def it(i, carry):
        off = base + i * (CH * NB)

        @pl.when(i > 0)
        def _():
            pltpu.make_async_copy(rows_v, f_out.at[pl.ds(0, CH * NB)],
                                  wb_sem).wait()
            pltpu.make_async_copy(aux_v, a_out.at[pl.ds(0, CH * NB * AUXO)],
                                  wb_sem).wait()

        cps = []
        for j in range(NB):
            cps.append(pltpu.async_copy(
                feats_hbm.at[idx_all.at[pl.ds((i * NB + j) * CH, CH)]],
                rows_v.at[pl.ds(j * CH, CH)], g_sem))

        for j in range(NB):
            for v in range(CH // LANES):
                idx16 = idx_all[pl.ds((i * NB + j) * CH + v * LANES,
                                      LANES)] * AUX
                dst = (lax.iota(jnp.int32, LANES)
                       + (j * CH + v * LANES)) * AUXO
                for c in range(AUX):
                    vals = plsc.load_gather(aux_tab, [idx16 + c])
                    plsc.store_scatter(aux_v, [dst + c], vals)

        for cp in cps:
            cp.wait()

        pltpu.async_copy(rows_v, f_out.at[pl.ds(off, CH * NB)], wb_sem)
        pltpu.async_copy(aux_v, a_out.at[pl.ds(off * AUXO, CH * NB * AUXO)],
                         wb_sem)
        return carry

    lax.fori_loop(0, N_IT, it, 0)
    pltpu.make_async_copy(rows_v, f_out.at[pl.ds(0, CH * NB)], wb_sem).wait()
    pltpu.make_async_copy(aux_v, a_out.at[pl.ds(0, CH * NB * AUXO)],
                          wb_sem).wait()


def _sc_gather(feats, aux_flat, idx_flat):
    mesh = plsc.VectorSubcoreMesh(core_axis_name="c", subcore_axis_name="s")
    return pl.kernel(
        _sc_gather_body,
        out_type=(
            jax.ShapeDtypeStruct((TOTAL_ROWS, IN_DIM), jnp.float32),
            jax.ShapeDtypeStruct((TOTAL_ROWS * AUXO,), jnp.float32),
        ),
        mesh=mesh,
        compiler_params=pltpu.CompilerParams(needs_layout_passes=False),
        scratch_types=[
            pltpu.VMEM((N_POINTS * AUX,), jnp.float32),
            pltpu.VMEM((PER_W,), jnp.int32),
            pltpu.VMEM((CH * NB, IN_DIM), jnp.float32),
            pltpu.VMEM((CH * NB * AUXO,), jnp.float32),
            pltpu.SemaphoreType.DMA,
            pltpu.SemaphoreType.DMA,
        ],
    )(feats, aux_flat, idx_flat)


def _tc_body(f_ref, aux_ref, qp_ref, kp_ref, w_ref, out_ref):
    nf = f_ref[...]
    qp = qp_ref[...]

    rel = aux_ref[...] - qp[:, None, :]
    rc = rel[:, :, :4]
    r2 = jnp.sum(rc * rc, axis=2, keepdims=True)

    cross = jax.lax.dot_general(
        rel, kp_ref[...],
        dimension_numbers=(((2,), (0,)), ((), ())),
        precision=jax.lax.Precision.HIGHEST,
        preferred_element_type=jnp.float32,
    )
    sq = jnp.maximum(r2 + cross, 0.0)
    wk = jnp.maximum(1.0 - jnp.sqrt(sq) * (1.0 / SIGMA), 0.0)

    wf = jax.lax.dot_general(
        wk, nf,
        dimension_numbers=(((1,), (1,)), ((0,), (0,))),
        preferred_element_type=jnp.float32,
    )

    ind = aux_ref[:, :, 4]
    cnt = jnp.maximum(jnp.sum(ind, axis=1, keepdims=True), 1.0)

    acc = jnp.dot(wf.reshape(BQ, KP * IN_DIM), w_ref[...],
                  preferred_element_type=jnp.float32)
    out_ref[...] = acc / cnt


@jax.jit
def _run(s_feats, aux_flat, q_points, idx_flat, weights_pad, kp_packed):
    f_g, a_g = _sc_gather(s_feats, aux_flat, idx_flat)
    f_g = f_g.reshape(N_POINTS, N_NEIGHBORS, IN_DIM)
    a_g = a_g.reshape(N_POINTS, N_NEIGHBORS, AUXO)

    grid = N_POINTS // BQ
    return pl.pallas_call(
        _tc_body,
        grid=(grid,),
        in_specs=[
            pl.BlockSpec((BQ, N_NEIGHBORS, IN_DIM), lambda i: (i, 0, 0)),
            pl.BlockSpec((BQ, N_NEIGHBORS, AUXO), lambda i: (i, 0, 0)),
            pl.BlockSpec((BQ, AUXO), lambda i: (i, 0)),
            pl.BlockSpec((AUXO, KP), lambda i: (0, 0)),
            pl.BlockSpec((KP * IN_DIM, OUT_DIM), lambda i: (0, 0)),
        ],
        out_specs=pl.BlockSpec((BQ, OUT_DIM), lambda i: (i, 0)),
        out_shape=jax.ShapeDtypeStruct((N_POINTS, OUT_DIM), jnp.float32),
    )(f_g, a_g, q_points, kp_packed, weights_pad)


def kernel(s_feats, q_points, s_points, neighbor_indices, weights, kernel_points):
    ones = jnp.ones((N_POINTS, 1), jnp.float32)
    ind = (jnp.sum(s_feats, axis=1, keepdims=True) > 0.0).astype(jnp.float32)
    aux = jnp.concatenate([s_points, ones, ind], axis=1)
    qp_pad = jnp.pad(q_points, ((0, 0), (0, AUXO - DIM)))
    kp_packed = jnp.zeros((AUXO, KP), jnp.float32)
    kp_packed = kp_packed.at[:3, :K_SIZE].set(-2.0 * kernel_points.T)
    kp_packed = kp_packed.at[3, :K_SIZE].set(
        jnp.sum(kernel_points ** 2, axis=1) - 1.0)
    kp_packed = kp_packed.at[3, K_SIZE].set(-1e9)
    weights_pad = jnp.concatenate(
        [weights, jnp.zeros((KP - K_SIZE, IN_DIM, OUT_DIM), jnp.float32)],
        axis=0).reshape(KP * IN_DIM, OUT_DIM)
    return _run(s_feats, aux.reshape(-1), qp_pad,
                neighbor_indices.reshape(-1), weights_pad, kp_packed)


# speedup vs baseline: 3.5468x; 1.1333x over previous
import functools

import jax
import jax.numpy as jnp
from jax import lax
from jax.experimental import pallas as pl
from jax.experimental.pallas import tpu as pltpu
from jax.experimental.pallas import tpu_sc as plsc

K_SIZE = 15
KP = 16
IN_DIM = 128
OUT_DIM = 128
SIGMA = 2.0
DIM = 3
N_POINTS = 10000
N_NEIGHBORS = 32
AUX = 5
AUXO = 8

BQ = 200

_SC_INFO = plsc.get_sparse_core_info()
NW = _SC_INFO.num_cores * _SC_INFO.num_subcores
TOTAL_ROWS = N_POINTS * N_NEIGHBORS
PER_W = TOTAL_ROWS // NW
CH = 80
NB = 5
N_IT = PER_W // (CH * NB)
CHUNKS_W = PER_W // CH
LANES = 16


def _sc_gather_body(feats_hbm, aux_hbm, idx_hbm, f_out, a_out,
                    aux_tab, idx_all, rows_v, aux_v, g_sem, wb_sem):
    wid = lax.axis_index("s") * _SC_INFO.num_cores + lax.axis_index("c")
    base = wid * PER_W

    pltpu.sync_copy(aux_hbm, aux_tab)
    pltpu.sync_copy(idx_hbm.at[pl.ds(base, PER_W)], idx_all)
    zero = jnp.zeros((LANES,), jnp.float32)
    for z in range(CH * NB * AUXO // LANES):
        aux_v[pl.ds(z * LANES, LANES)] = zero

    def it(i, carry):
        off = base + i * (CH * NB)

        @pl.when(i > 0)
        def _():
            pltpu.make_async_copy(rows_v, f_out.at[pl.ds(0, CH * NB)],
                                  wb_sem).wait()
            pltpu.make_async_copy(aux_v, a_out.at[pl.ds(0, CH * NB * AUXO)],
                                  wb_sem).wait()

        cps = []
        for j in range(NB):
            cps.append(pltpu.async_copy(
                feats_hbm.at[idx_all.at[pl.ds((i * NB + j) * CH, CH)]],
                rows_v.at[pl.ds(j * CH, CH)], g_sem))

        for j in range(NB):
            for v in range(CH // LANES):
                idx16 = idx_all[pl.ds((i * NB + j) * CH + v * LANES,
                                      LANES)] * AUX
                dst = (lax.iota(jnp.int32, LANES)
                       + (j * CH + v * LANES)) * AUXO
                for c in range(AUX):
                    vals = plsc.load_gather(aux_tab, [idx16 + c])
                    plsc.store_scatter(aux_v, [dst + c], vals)

        for cp in cps:
            cp.wait()

        pltpu.async_copy(rows_v, f_out.at[pl.ds(off, CH * NB)], wb_sem)
        pltpu.async_copy(aux_v, a_out.at[pl.ds(off * AUXO, CH * NB * AUXO)],
                         wb_sem)
        return carry

    lax.fori_loop(0, N_IT, it, 0)
    pltpu.make_async_copy(rows_v, f_out.at[pl.ds(0, CH * NB)], wb_sem).wait()
    pltpu.make_async_copy(aux_v, a_out.at[pl.ds(0, CH * NB * AUXO)],
                          wb_sem).wait()


def _sc_gather(feats, aux_flat, idx_flat):
    mesh = plsc.VectorSubcoreMesh(core_axis_name="c", subcore_axis_name="s")
    return pl.kernel(
        _sc_gather_body,
        out_type=(
            jax.ShapeDtypeStruct((TOTAL_ROWS, IN_DIM), jnp.float32),
            jax.ShapeDtypeStruct((TOTAL_ROWS * AUXO,), jnp.float32),
        ),
        mesh=mesh,
        compiler_params=pltpu.CompilerParams(needs_layout_passes=False),
        scratch_types=[
            pltpu.VMEM((N_POINTS * AUX,), jnp.float32),
            pltpu.VMEM((PER_W,), jnp.int32),
            pltpu.VMEM((CH * NB, IN_DIM), jnp.float32),
            pltpu.VMEM((CH * NB * AUXO,), jnp.float32),
            pltpu.SemaphoreType.DMA,
            pltpu.SemaphoreType.DMA,
        ],
    )(feats, aux_flat, idx_flat)


def _tc_body(f_ref, aux_ref, qp_ref, kph_ref, kpl_ref, w_ref, out_ref):
    nf = f_ref[...]
    qp = qp_ref[...]

    rel = aux_ref[...] - qp[:, None, :]
    rc = rel[:, :, :4]
    r2 = jnp.sum(rc * rc, axis=2, keepdims=True)

    dn = (((2,), (0,)), ((0,), ()))
    dn = (((2,), (0,)), ((), ()))
    relh = rel.astype(jnp.bfloat16)
    rell = (rel - relh.astype(jnp.float32)).astype(jnp.bfloat16)
    kph = kph_ref[...]
    kpl = kpl_ref[...]
    cross = (
        jax.lax.dot_general(relh, kph, dimension_numbers=dn,
                            preferred_element_type=jnp.float32)
        + jax.lax.dot_general(relh, kpl, dimension_numbers=dn,
                              preferred_element_type=jnp.float32)
        + jax.lax.dot_general(rell, kph, dimension_numbers=dn,
                              preferred_element_type=jnp.float32)
    )
    sq = jnp.maximum(r2 + cross, 0.0)
    wk = jnp.maximum(1.0 - jnp.sqrt(sq) * (1.0 / SIGMA), 0.0)

    wf = jax.lax.dot_general(
        wk, nf,
        dimension_numbers=(((1,), (1,)), ((0,), (0,))),
        preferred_element_type=jnp.float32,
    )

    ind = aux_ref[:, :, 4]
    cnt = jnp.maximum(jnp.sum(ind, axis=1, keepdims=True), 1.0)

    acc = jnp.dot(wf.reshape(BQ, KP * IN_DIM), w_ref[...],
                  preferred_element_type=jnp.float32)
    out_ref[...] = acc / cnt


@jax.jit
def _run(s_feats, aux_flat, q_points, idx_flat, weights_pad, kph, kpl):
    f_g, a_g = _sc_gather(s_feats, aux_flat, idx_flat)
    f_g = f_g.reshape(N_POINTS, N_NEIGHBORS, IN_DIM)
    a_g = a_g.reshape(N_POINTS, N_NEIGHBORS, AUXO)

    grid = N_POINTS // BQ
    return pl.pallas_call(
        _tc_body,
        grid=(grid,),
        in_specs=[
            pl.BlockSpec((BQ, N_NEIGHBORS, IN_DIM), lambda i: (i, 0, 0)),
            pl.BlockSpec((BQ, N_NEIGHBORS, AUXO), lambda i: (i, 0, 0)),
            pl.BlockSpec((BQ, AUXO), lambda i: (i, 0)),
            pl.BlockSpec((AUXO, KP), lambda i: (0, 0)),
            pl.BlockSpec((AUXO, KP), lambda i: (0, 0)),
            pl.BlockSpec((KP * IN_DIM, OUT_DIM), lambda i: (0, 0)),
        ],
        out_specs=pl.BlockSpec((BQ, OUT_DIM), lambda i: (i, 0)),
        out_shape=jax.ShapeDtypeStruct((N_POINTS, OUT_DIM), jnp.float32),
    )(f_g, a_g, q_points, kph, kpl, weights_pad)


def kernel(s_feats, q_points, s_points, neighbor_indices, weights, kernel_points):
    ones = jnp.ones((N_POINTS, 1), jnp.float32)
    ind = (jnp.sum(s_feats, axis=1, keepdims=True) > 0.0).astype(jnp.float32)
    aux = jnp.concatenate([s_points, ones, ind], axis=1)
    qp_pad = jnp.pad(q_points, ((0, 0), (0, AUXO - DIM)))
    kp_packed = jnp.zeros((AUXO, KP), jnp.float32)
    kp_packed = kp_packed.at[:3, :K_SIZE].set(-2.0 * kernel_points.T)
    kp_packed = kp_packed.at[3, :K_SIZE].set(
        jnp.sum(kernel_points ** 2, axis=1) - 1.0)
    kp_packed = kp_packed.at[3, K_SIZE].set(-1e9)
    kph = kp_packed.astype(jnp.bfloat16)
    kpl = (kp_packed - kph.astype(jnp.float32)).astype(jnp.bfloat16)
    weights_pad = jnp.concatenate(
        [weights, jnp.zeros((KP - K_SIZE, IN_DIM, OUT_DIM), jnp.float32)],
        axis=0).reshape(KP * IN_DIM, OUT_DIM)
    return _run(s_feats, aux.reshape(-1), qp_pad,
                neighbor_indices.reshape(-1), weights_pad, kph, kpl)
